# baseline (speedup 1.0000x reference)
import functools

import jax
import jax.numpy as jnp
from jax import lax
from jax.experimental import pallas as pl
from jax.experimental.pallas import tpu as pltpu
from jax.experimental.pallas import tpu_sc as plsc

N = 10000
NP = 10240
E = 320000
EP = 327680
F = 128
HID = 128
CLS = 64
NC = 2
NS = 16
K = 128
DCH = EP // (NC * NS * K)
RPT = NP // NS
RB = 2048
GRID = NP // RB

f32 = jnp.float32

_mesh = plsc.VectorSubcoreMesh(
    core_axis_name="c", subcore_axis_name="s", num_cores=NC, num_subcores=NS)

_sc_untiled = pltpu.CompilerParams(use_tc_tiling_on_sc=False)


def _zero_fill(ref, rows, width):
  @pl.loop(0, rows)
  def _(i):
    for j in range(width // 16):
      ref[i, pl.ds(j * 16, 16)] = jnp.zeros((16,), f32)


def _degree_pass(src2, dst2):
  @functools.partial(
      pl.kernel, mesh=_mesh, compiler_params=_sc_untiled,
      out_type=(jax.ShapeDtypeStruct((NC, NP), f32),
                jax.ShapeDtypeStruct((NC, NP), f32)),
      scratch_types=[
          pltpu.VMEM_SHARED((NP,), f32),
          pltpu.VMEM_SHARED((NP,), f32),
          pltpu.VMEM((DCH, K), jnp.int32),
          pltpu.VMEM((DCH, K), jnp.int32),
          pltpu.VMEM((K,), f32),
          pltpu.VMEM((RPT,), f32),
          pltpu.SemaphoreType.DMA,
      ])
  def body(src_h, dst_h, do_h, di_h, do_sh, di_sh, sidx, didx, ones_v, zb,
           sm):
    c = lax.axis_index("c")
    s = lax.axis_index("s")
    w = c * NS + s
    pltpu.sync_copy(src_h.at[w], sidx)
    pltpu.sync_copy(dst_h.at[w], didx)

    @pl.loop(0, RPT // 16)
    def _(i):
      zb[pl.ds(i * 16, 16)] = jnp.zeros((16,), f32)
    for j in range(K // 16):
      ones_v[pl.ds(j * 16, 16)] = jnp.ones((16,), f32)
    pltpu.sync_copy(zb, do_sh.at[pl.ds(s * RPT, RPT)])
    pltpu.sync_copy(zb, di_sh.at[pl.ds(s * RPT, RPT)])
    plsc.subcore_barrier()

    @pl.loop(0, DCH, step=4)
    def _(j):
      for t in range(4):
        pltpu.async_copy(ones_v, do_sh.at[sidx.at[j + t]], sm, add=True)
        pltpu.async_copy(ones_v, di_sh.at[didx.at[j + t]], sm, add=True)
      for t in range(4):
        pltpu.make_async_copy(ones_v, do_sh.at[sidx.at[j + t]], sm).wait()
        pltpu.make_async_copy(ones_v, di_sh.at[didx.at[j + t]], sm).wait()

    plsc.subcore_barrier()
    pltpu.sync_copy(do_sh.at[pl.ds(s * RPT, RPT)], do_h.at[c, pl.ds(s * RPT, RPT)])
    pltpu.sync_copy(di_sh.at[pl.ds(s * RPT, RPT)], di_h.at[c, pl.ds(s * RPT, RPT)])

  return body(src2, dst2)


def _mp(xw, src2, dst2, dh, tiled, nph):
  pch = DCH // nph

  @functools.partial(
      pl.kernel, mesh=_mesh,
      compiler_params=None if tiled else _sc_untiled,
      out_type=jax.ShapeDtypeStruct((NC, NP, dh), f32),
      scratch_types=[
          pltpu.VMEM_SHARED((NP, dh), f32),
          pltpu.VMEM((pch, K), jnp.int32),
          pltpu.VMEM((pch, K), jnp.int32),
          pltpu.VMEM((K, dh), f32),
          pltpu.VMEM((K, dh), f32),
          pltpu.SemaphoreType.DMA,
          pltpu.SemaphoreType.DMA,
          pltpu.SemaphoreType.DMA,
          pltpu.SemaphoreType.DMA,
      ])
  def body(xw_h, src_h, dst_h, out_h, acc, sidx, didx, r0, r1, sm0, sm1,
           tm0, tm1):
    c = lax.axis_index("c")
    s = lax.axis_index("s")
    w = c * NS + s
    _zero_fill(r0, K, dh)
    for k in range(RPT // K):
      pltpu.sync_copy(r0, acc.at[pl.ds(s * RPT + k * K, K)])
    plsc.subcore_barrier()

    for ph in range(nph):
      pltpu.sync_copy(src_h.at[w, pl.ds(ph * pch, pch)], sidx)
      pltpu.sync_copy(dst_h.at[w, pl.ds(ph * pch, pch)], didx)
      pltpu.async_copy(xw_h.at[sidx.at[0]], r0, sm0)
      pltpu.async_copy(xw_h.at[sidx.at[1]], r1, sm1)

      @pl.loop(0, pch, step=2)
      def _(g):
        pltpu.make_async_copy(xw_h.at[sidx.at[g]], r0, sm0).wait()
        pltpu.async_copy(r0, acc.at[didx.at[g]], tm0, add=True)
        pltpu.make_async_copy(xw_h.at[sidx.at[g + 1]], r1, sm1).wait()
        pltpu.async_copy(r1, acc.at[didx.at[g + 1]], tm1, add=True)

        @pl.when(g + 2 < pch)
        def _():
          pltpu.make_async_copy(r0, acc.at[didx.at[g]], tm0).wait()
          pltpu.async_copy(xw_h.at[sidx.at[g + 2]], r0, sm0)
          pltpu.make_async_copy(r1, acc.at[didx.at[g + 1]], tm1).wait()
          pltpu.async_copy(xw_h.at[sidx.at[g + 3]], r1, sm1)

      pltpu.make_async_copy(r0, acc.at[didx.at[pch - 2]], tm0).wait()
      pltpu.make_async_copy(r1, acc.at[didx.at[pch - 1]], tm1).wait()

    plsc.subcore_barrier()
    pltpu.sync_copy(acc.at[pl.ds(s * RPT, RPT)], out_h.at[c, pl.ds(s * RPT, RPT)])

  return body(xw, src2, dst2)


def _norm(dref):
  deg = dref[0] + dref[1]
  return lax.rsqrt(jnp.maximum(deg, 1.0))[:, None]


def _m1(x_p, W1, dop):
  def body(x_ref, w_ref, d_ref, o_ref):
    xw = jnp.dot(x_ref[...], w_ref[...], preferred_element_type=f32)
    o_ref[...] = xw * _norm(d_ref)

  return pl.pallas_call(
      body, grid=(GRID,),
      in_specs=[
          pl.BlockSpec((RB, F), lambda i: (i, 0)),
          pl.BlockSpec((F, HID), lambda i: (0, 0)),
          pl.BlockSpec((NC, RB), lambda i: (0, i)),
      ],
      out_specs=pl.BlockSpec((RB, HID), lambda i: (i, 0)),
      out_shape=jax.ShapeDtypeStruct((NP, HID), f32))(x_p, W1, dop)


def _m2(agg1, dop, dip, b1, W2):
  def body(a_ref, do_ref, di_ref, b_ref, w_ref, o_ref):
    agg = a_ref[0] + a_ref[1]
    h = jnp.maximum(agg * _norm(di_ref) + b_ref[...], 0.0)
    h = h * _norm(do_ref)
    o_ref[...] = jnp.dot(h, w_ref[...], preferred_element_type=f32)

  return pl.pallas_call(
      body, grid=(GRID,),
      in_specs=[
          pl.BlockSpec((NC, RB, HID), lambda i: (0, i, 0)),
          pl.BlockSpec((NC, RB), lambda i: (0, i)),
          pl.BlockSpec((NC, RB), lambda i: (0, i)),
          pl.BlockSpec((1, HID), lambda i: (0, 0)),
          pl.BlockSpec((HID, CLS), lambda i: (0, 0)),
      ],
      out_specs=pl.BlockSpec((RB, CLS), lambda i: (i, 0)),
      out_shape=jax.ShapeDtypeStruct((NP, CLS), f32))(agg1, dop, dip, b1, W2)


def _m3(agg2, dip, b2):
  def body(a_ref, di_ref, b_ref, o_ref):
    z = a_ref[0] + a_ref[1]
    z = z * _norm(di_ref) + b_ref[...]
    z = z - jnp.max(z, axis=1, keepdims=True)
    ez = jnp.exp(z)
    o_ref[...] = ez / jnp.sum(ez, axis=1, keepdims=True)

  return pl.pallas_call(
      body, grid=(GRID,),
      in_specs=[
          pl.BlockSpec((NC, RB, CLS), lambda i: (0, i, 0)),
          pl.BlockSpec((NC, RB), lambda i: (0, i)),
          pl.BlockSpec((1, CLS), lambda i: (0, 0)),
      ],
      out_specs=pl.BlockSpec((RB, CLS), lambda i: (i, 0)),
      out_shape=jax.ShapeDtypeStruct((NP, CLS), f32))(agg2, dip, b2)


def kernel(x, edge_index, W1, b1, W2, b2):
  pad = 10000 + (jnp.arange(EP - E, dtype=jnp.int32) % (NP - N))
  src2 = jnp.concatenate([edge_index[0], pad]).reshape(NC * NS, DCH, K)
  dst2 = jnp.concatenate([edge_index[1], pad]).reshape(NC * NS, DCH, K)

  x_p = jnp.pad(x, ((0, NP - N), (0, 0)))
  dop, dip = _degree_pass(src2, dst2)

  xw1 = _m1(x_p, W1, dop)
  agg1 = _mp(xw1, src2, dst2, HID, True, 2)
  xw2 = _m2(agg1, dop, dip, b1.reshape(1, HID), W2)
  agg2 = _mp(xw2, src2, dst2, CLS, False, 1)
  out = _m3(agg2, dip, b2.reshape(1, CLS))
  return out[:N]


# speedup vs baseline: 1.2049x; 1.2049x over previous
import functools

import jax
import jax.numpy as jnp
from jax import lax
from jax.experimental import pallas as pl
from jax.experimental.pallas import tpu as pltpu
from jax.experimental.pallas import tpu_sc as plsc

N = 10000
NP = 10240
E = 320000
EP = 327680
F = 128
HID = 128
CLS = 64
NC = 2
NS = 16
K = 128
DCH = EP // (NC * NS * K)
RPT = NP // NS
RB = 2048
GRID = NP // RB

f32 = jnp.float32

_mesh = plsc.VectorSubcoreMesh(
    core_axis_name="c", subcore_axis_name="s", num_cores=NC, num_subcores=NS)

_sc_untiled = pltpu.CompilerParams(use_tc_tiling_on_sc=False)


def _zero_fill(ref, rows, width):
  @pl.loop(0, rows)
  def _(i):
    for j in range(width // 16):
      ref[i, pl.ds(j * 16, 16)] = jnp.zeros((16,), f32)


def _degree_pass(src2, dst2):
  @functools.partial(
      pl.kernel, mesh=_mesh, compiler_params=_sc_untiled,
      out_type=(jax.ShapeDtypeStruct((NC, NP), f32),
                jax.ShapeDtypeStruct((NC, NP), f32)),
      scratch_types=[
          pltpu.VMEM_SHARED((NP,), f32),
          pltpu.VMEM_SHARED((NP,), f32),
          pltpu.VMEM((DCH, K), jnp.int32),
          pltpu.VMEM((DCH, K), jnp.int32),
          pltpu.VMEM((K,), f32),
          pltpu.VMEM((RPT,), f32),
          pltpu.SemaphoreType.DMA,
      ])
  def body(src_h, dst_h, do_h, di_h, do_sh, di_sh, sidx, didx, ones_v, zb,
           sm):
    c = lax.axis_index("c")
    s = lax.axis_index("s")
    w = c * NS + s
    pltpu.sync_copy(src_h.at[w], sidx)
    pltpu.sync_copy(dst_h.at[w], didx)

    @pl.loop(0, RPT // 16)
    def _(i):
      zb[pl.ds(i * 16, 16)] = jnp.zeros((16,), f32)
    for j in range(K // 16):
      ones_v[pl.ds(j * 16, 16)] = jnp.ones((16,), f32)
    pltpu.sync_copy(zb, do_sh.at[pl.ds(s * RPT, RPT)])
    pltpu.sync_copy(zb, di_sh.at[pl.ds(s * RPT, RPT)])
    plsc.subcore_barrier()

    @pl.loop(0, DCH, step=4)
    def _(j):
      for t in range(4):
        pltpu.async_copy(ones_v, do_sh.at[sidx.at[j + t]], sm, add=True)
        pltpu.async_copy(ones_v, di_sh.at[didx.at[j + t]], sm, add=True)
      for t in range(4):
        pltpu.make_async_copy(ones_v, do_sh.at[sidx.at[j + t]], sm).wait()
        pltpu.make_async_copy(ones_v, di_sh.at[didx.at[j + t]], sm).wait()

    plsc.subcore_barrier()
    pltpu.sync_copy(do_sh.at[pl.ds(s * RPT, RPT)], do_h.at[c, pl.ds(s * RPT, RPT)])
    pltpu.sync_copy(di_sh.at[pl.ds(s * RPT, RPT)], di_h.at[c, pl.ds(s * RPT, RPT)])

  return body(src2, dst2)


def _mp(xw, src2, dst2, dh, tiled, nph):
  pch = DCH // nph

  @functools.partial(
      pl.kernel, mesh=_mesh,
      compiler_params=None if tiled else _sc_untiled,
      out_type=jax.ShapeDtypeStruct((NC, NP, dh), f32),
      scratch_types=[
          pltpu.VMEM_SHARED((NP, dh), f32),
          pltpu.VMEM((pch, K), jnp.int32),
          pltpu.VMEM((pch, K), jnp.int32),
          pltpu.VMEM((K, dh), f32),
          pltpu.VMEM((K, dh), f32),
          pltpu.SemaphoreType.DMA,
          pltpu.SemaphoreType.DMA,
      ])
  def body(xw_h, src_h, dst_h, out_h, acc, sidx, didx, r0, r1, sm0, sm1):
    c = lax.axis_index("c")
    s = lax.axis_index("s")
    w = c * NS + s
    _zero_fill(r0, K, dh)
    for k in range(RPT // K):
      pltpu.sync_copy(r0, acc.at[pl.ds(s * RPT + k * K, K)])
    plsc.subcore_barrier()

    for ph in range(nph):
      pltpu.sync_copy(src_h.at[w, pl.ds(ph * pch, pch)], sidx)
      pltpu.sync_copy(dst_h.at[w, pl.ds(ph * pch, pch)], didx)
      pltpu.async_copy(xw_h.at[sidx.at[0]], r0, sm0)

      @pl.loop(0, pch, step=2)
      def _(g):
        pltpu.async_copy(xw_h.at[sidx.at[g + 1]], r1, sm1)
        pltpu.make_async_copy(xw_h.at[sidx.at[g]], r0, sm0).wait()
        pltpu.sync_copy(r0, acc.at[didx.at[g]], add=True)

        @pl.when(g + 2 < pch)
        def _():
          pltpu.async_copy(xw_h.at[sidx.at[g + 2]], r0, sm0)

        pltpu.make_async_copy(xw_h.at[sidx.at[g + 1]], r1, sm1).wait()
        pltpu.sync_copy(r1, acc.at[didx.at[g + 1]], add=True)

    plsc.subcore_barrier()
    pltpu.sync_copy(acc.at[pl.ds(s * RPT, RPT)], out_h.at[c, pl.ds(s * RPT, RPT)])

  return body(xw, src2, dst2)


def _norm(dref):
  deg = dref[0] + dref[1]
  return lax.rsqrt(jnp.maximum(deg, 1.0))[:, None]


def _m1(x_p, W1, dop):
  def body(x_ref, w_ref, d_ref, o_ref):
    xw = jnp.dot(x_ref[...], w_ref[...], preferred_element_type=f32)
    o_ref[...] = xw * _norm(d_ref)

  return pl.pallas_call(
      body, grid=(GRID,),
      in_specs=[
          pl.BlockSpec((RB, F), lambda i: (i, 0)),
          pl.BlockSpec((F, HID), lambda i: (0, 0)),
          pl.BlockSpec((NC, RB), lambda i: (0, i)),
      ],
      out_specs=pl.BlockSpec((RB, HID), lambda i: (i, 0)),
      out_shape=jax.ShapeDtypeStruct((NP, HID), f32))(x_p, W1, dop)


def _m2(agg1, dop, dip, b1, W2):
  def body(a_ref, do_ref, di_ref, b_ref, w_ref, o_ref):
    agg = a_ref[0] + a_ref[1]
    h = jnp.maximum(agg * _norm(di_ref) + b_ref[...], 0.0)
    h = h * _norm(do_ref)
    o_ref[...] = jnp.dot(h, w_ref[...], preferred_element_type=f32)

  return pl.pallas_call(
      body, grid=(GRID,),
      in_specs=[
          pl.BlockSpec((NC, RB, HID), lambda i: (0, i, 0)),
          pl.BlockSpec((NC, RB), lambda i: (0, i)),
          pl.BlockSpec((NC, RB), lambda i: (0, i)),
          pl.BlockSpec((1, HID), lambda i: (0, 0)),
          pl.BlockSpec((HID, CLS), lambda i: (0, 0)),
      ],
      out_specs=pl.BlockSpec((RB, CLS), lambda i: (i, 0)),
      out_shape=jax.ShapeDtypeStruct((NP, CLS), f32))(agg1, dop, dip, b1, W2)


def _m3(agg2, dip, b2):
  def body(a_ref, di_ref, b_ref, o_ref):
    z = a_ref[0] + a_ref[1]
    z = z * _norm(di_ref) + b_ref[...]
    z = z - jnp.max(z, axis=1, keepdims=True)
    ez = jnp.exp(z)
    o_ref[...] = ez / jnp.sum(ez, axis=1, keepdims=True)

  return pl.pallas_call(
      body, grid=(GRID,),
      in_specs=[
          pl.BlockSpec((NC, RB, CLS), lambda i: (0, i, 0)),
          pl.BlockSpec((NC, RB), lambda i: (0, i)),
          pl.BlockSpec((1, CLS), lambda i: (0, 0)),
      ],
      out_specs=pl.BlockSpec((RB, CLS), lambda i: (i, 0)),
      out_shape=jax.ShapeDtypeStruct((NP, CLS), f32))(agg2, dip, b2)


def kernel(x, edge_index, W1, b1, W2, b2):
  pad = 10000 + (jnp.arange(EP - E, dtype=jnp.int32) % (NP - N))
  src2 = jnp.concatenate([edge_index[0], pad]).reshape(NC * NS, DCH, K)
  dst2 = jnp.concatenate([edge_index[1], pad]).reshape(NC * NS, DCH, K)

  x_p = jnp.pad(x, ((0, NP - N), (0, 0)))
  dop, dip = _degree_pass(src2, dst2)

  xw1 = _m1(x_p, W1, dop)
  agg1 = _mp(xw1, src2, dst2, HID, True, 2)
  xw2 = _m2(agg1, dop, dip, b1.reshape(1, HID), W2)
  agg2 = _mp(xw2, src2, dst2, CLS, False, 1)
  out = _m3(agg2, dip, b2.reshape(1, CLS))
  return out[:N]


# speedup vs baseline: 1.2358x; 1.0256x over previous
import functools

import jax
import jax.numpy as jnp
from jax import lax
from jax.experimental import pallas as pl
from jax.experimental.pallas import tpu as pltpu
from jax.experimental.pallas import tpu_sc as plsc

N = 10000
NP = 10240
E = 320000
EP = 327680
F = 128
HID = 128
CLS = 64
NC = 2
NS = 16
K = 128
DCH = EP // (NC * NS * K)
RPT = NP // NS
RB = 2048
GRID = NP // RB

f32 = jnp.float32

_mesh = plsc.VectorSubcoreMesh(
    core_axis_name="c", subcore_axis_name="s", num_cores=NC, num_subcores=NS)

_sc_untiled = pltpu.CompilerParams(use_tc_tiling_on_sc=False)


def _zero_fill(ref, rows, width):
  @pl.loop(0, rows)
  def _(i):
    for j in range(width // 16):
      ref[i, pl.ds(j * 16, 16)] = jnp.zeros((16,), f32)


def _degree_pass(src2, dst2):
  @functools.partial(
      pl.kernel, mesh=_mesh, compiler_params=_sc_untiled,
      out_type=(jax.ShapeDtypeStruct((NC, NP), f32),
                jax.ShapeDtypeStruct((NC, NP), f32)),
      scratch_types=[
          pltpu.VMEM_SHARED((NP,), f32),
          pltpu.VMEM_SHARED((NP,), f32),
          pltpu.VMEM((DCH, K), jnp.int32),
          pltpu.VMEM((DCH, K), jnp.int32),
          pltpu.VMEM((K,), f32),
          pltpu.VMEM((RPT,), f32),
          pltpu.SemaphoreType.DMA,
      ])
  def body(src_h, dst_h, do_h, di_h, do_sh, di_sh, sidx, didx, ones_v, zb,
           sm):
    c = lax.axis_index("c")
    s = lax.axis_index("s")
    w = c * NS + s
    pltpu.sync_copy(src_h.at[w], sidx)
    pltpu.sync_copy(dst_h.at[w], didx)

    @pl.loop(0, RPT // 16)
    def _(i):
      zb[pl.ds(i * 16, 16)] = jnp.zeros((16,), f32)
    for j in range(K // 16):
      ones_v[pl.ds(j * 16, 16)] = jnp.ones((16,), f32)
    pltpu.sync_copy(zb, do_sh.at[pl.ds(s * RPT, RPT)])
    pltpu.sync_copy(zb, di_sh.at[pl.ds(s * RPT, RPT)])
    plsc.subcore_barrier()

    @pl.loop(0, DCH, step=4)
    def _(j):
      for t in range(4):
        pltpu.async_copy(ones_v, do_sh.at[sidx.at[j + t]], sm, add=True)
        pltpu.async_copy(ones_v, di_sh.at[didx.at[j + t]], sm, add=True)
      for t in range(4):
        pltpu.make_async_copy(ones_v, do_sh.at[sidx.at[j + t]], sm).wait()
        pltpu.make_async_copy(ones_v, di_sh.at[didx.at[j + t]], sm).wait()

    plsc.subcore_barrier()
    pltpu.sync_copy(do_sh.at[pl.ds(s * RPT, RPT)], do_h.at[c, pl.ds(s * RPT, RPT)])
    pltpu.sync_copy(di_sh.at[pl.ds(s * RPT, RPT)], di_h.at[c, pl.ds(s * RPT, RPT)])

  return body(src2, dst2)


def _mp(xw, src2, dst2, dh, tiled, nph, nbuf=2):
  pch = DCH // nph

  @functools.partial(
      pl.kernel, mesh=_mesh,
      compiler_params=None if tiled else _sc_untiled,
      out_type=jax.ShapeDtypeStruct((NC, NP, dh), f32),
      scratch_types=[
          pltpu.VMEM_SHARED((NP, dh), f32),
          pltpu.VMEM((pch, K), jnp.int32),
          pltpu.VMEM((pch, K), jnp.int32),
      ] + [pltpu.VMEM((K, dh), f32)] * nbuf
        + [pltpu.SemaphoreType.DMA] * (2 * nbuf))
  def body(xw_h, src_h, dst_h, out_h, acc, sidx, didx, *bufs_sems):
    r = bufs_sems[:nbuf]
    gm = bufs_sems[nbuf:2 * nbuf]
    tm = bufs_sems[2 * nbuf:]
    c = lax.axis_index("c")
    s = lax.axis_index("s")
    w = c * NS + s
    _zero_fill(r[0], K, dh)
    for k in range(RPT // K):
      pltpu.sync_copy(r[0], acc.at[pl.ds(s * RPT + k * K, K)])
    plsc.subcore_barrier()

    for ph in range(nph):
      pltpu.sync_copy(src_h.at[w, pl.ds(ph * pch, pch)], sidx)
      pltpu.sync_copy(dst_h.at[w, pl.ds(ph * pch, pch)], didx)

      if nbuf == 2:
        pltpu.async_copy(xw_h.at[sidx.at[0]], r[0], gm[0])

        @pl.loop(0, pch, step=2)
        def _(g):
          pltpu.async_copy(xw_h.at[sidx.at[g + 1]], r[1], gm[1])
          pltpu.make_async_copy(xw_h.at[sidx.at[g]], r[0], gm[0]).wait()
          pltpu.sync_copy(r[0], acc.at[didx.at[g]], add=True)

          @pl.when(g + 2 < pch)
          def _():
            pltpu.async_copy(xw_h.at[sidx.at[g + 2]], r[0], gm[0])

          pltpu.make_async_copy(xw_h.at[sidx.at[g + 1]], r[1], gm[1]).wait()
          pltpu.sync_copy(r[1], acc.at[didx.at[g + 1]], add=True)
      else:
        for b in range(nbuf):
          pltpu.async_copy(xw_h.at[sidx.at[b]], r[b], gm[b])

        @pl.loop(0, pch, step=nbuf)
        def _(g):
          for b in range(nbuf):
            pltpu.make_async_copy(xw_h.at[sidx.at[g + b]], r[b], gm[b]).wait()
            pltpu.async_copy(r[b], acc.at[didx.at[g + b]], tm[b], add=True)
          for b in range(nbuf):
            @pl.when(g + nbuf + b < pch)
            def _(b=b):
              pltpu.make_async_copy(r[b], acc.at[didx.at[g + b]], tm[b]).wait()
              pltpu.async_copy(xw_h.at[sidx.at[g + nbuf + b]], r[b], gm[b])

        for b in range(nbuf):
          pltpu.make_async_copy(
              r[b], acc.at[didx.at[pch - nbuf + b]], tm[b]).wait()

    plsc.subcore_barrier()
    pltpu.sync_copy(acc.at[pl.ds(s * RPT, RPT)], out_h.at[c, pl.ds(s * RPT, RPT)])

  return body(xw, src2, dst2)


def _norm(dref):
  deg = dref[0] + dref[1]
  return lax.rsqrt(jnp.maximum(deg, 1.0))[:, None]


def _m1(x_p, W1, dop):
  def body(x_ref, w_ref, d_ref, o_ref):
    xw = jnp.dot(x_ref[...], w_ref[...], preferred_element_type=f32)
    o_ref[...] = xw * _norm(d_ref)

  return pl.pallas_call(
      body, grid=(GRID,),
      in_specs=[
          pl.BlockSpec((RB, F), lambda i: (i, 0)),
          pl.BlockSpec((F, HID), lambda i: (0, 0)),
          pl.BlockSpec((NC, RB), lambda i: (0, i)),
      ],
      out_specs=pl.BlockSpec((RB, HID), lambda i: (i, 0)),
      out_shape=jax.ShapeDtypeStruct((NP, HID), f32))(x_p, W1, dop)


def _m2(agg1, dop, dip, b1, W2):
  def body(a_ref, do_ref, di_ref, b_ref, w_ref, o_ref):
    agg = a_ref[0] + a_ref[1]
    h = jnp.maximum(agg * _norm(di_ref) + b_ref[...], 0.0)
    h = h * _norm(do_ref)
    o_ref[...] = jnp.dot(h, w_ref[...], preferred_element_type=f32)

  return pl.pallas_call(
      body, grid=(GRID,),
      in_specs=[
          pl.BlockSpec((NC, RB, HID), lambda i: (0, i, 0)),
          pl.BlockSpec((NC, RB), lambda i: (0, i)),
          pl.BlockSpec((NC, RB), lambda i: (0, i)),
          pl.BlockSpec((1, HID), lambda i: (0, 0)),
          pl.BlockSpec((HID, CLS), lambda i: (0, 0)),
      ],
      out_specs=pl.BlockSpec((RB, CLS), lambda i: (i, 0)),
      out_shape=jax.ShapeDtypeStruct((NP, CLS), f32))(agg1, dop, dip, b1, W2)


def _m3(agg2, dip, b2):
  def body(a_ref, di_ref, b_ref, o_ref):
    z = a_ref[0] + a_ref[1]
    z = z * _norm(di_ref) + b_ref[...]
    z = z - jnp.max(z, axis=1, keepdims=True)
    ez = jnp.exp(z)
    o_ref[...] = ez / jnp.sum(ez, axis=1, keepdims=True)

  return pl.pallas_call(
      body, grid=(GRID,),
      in_specs=[
          pl.BlockSpec((NC, RB, CLS), lambda i: (0, i, 0)),
          pl.BlockSpec((NC, RB), lambda i: (0, i)),
          pl.BlockSpec((1, CLS), lambda i: (0, 0)),
      ],
      out_specs=pl.BlockSpec((RB, CLS), lambda i: (i, 0)),
      out_shape=jax.ShapeDtypeStruct((NP, CLS), f32))(agg2, dip, b2)


def kernel(x, edge_index, W1, b1, W2, b2):
  pad = 10000 + (jnp.arange(EP - E, dtype=jnp.int32) % (NP - N))
  src2 = jnp.concatenate([edge_index[0], pad]).reshape(NC * NS, DCH, K)
  dst2 = jnp.concatenate([edge_index[1], pad]).reshape(NC * NS, DCH, K)

  x_p = jnp.pad(x, ((0, NP - N), (0, 0)))
  dop, dip = _degree_pass(src2, dst2)

  xw1 = _m1(x_p, W1, dop)
  agg1 = _mp(xw1, src2, dst2, HID, True, 2)
  xw2 = _m2(agg1, dop, dip, b1.reshape(1, HID), W2)
  agg2 = _mp(xw2, src2, dst2, CLS, False, 1, nbuf=4)
  out = _m3(agg2, dip, b2.reshape(1, CLS))
  return out[:N]
